# Initial kernel scaffold; baseline (speedup 1.0000x reference)
import jax
import jax.numpy as jnp
from jax.experimental import pallas as pl


def kernel(x, edge_index, w, W0, b0, W1, b1, Wp, bp, Wv, bv):
    raise NotImplementedError("write your pallas kernel here")


# speedup vs baseline: 3.6773x; 3.6773x over previous
import dataclasses
import functools

import jax
import jax.numpy as jnp
from jax import lax
from jax.experimental import pallas as pl
from jax.experimental.pallas import tpu as pltpu
from jax.experimental.pallas import tpu_sc as plsc

N = 10000
E = 320000
D = 128
NPAD = 10240
NC = 2
NS = 16
K = 80
EPT = E // (NC * NS)
CH = EPT // K
EPT1 = E // NS
CH1 = EPT1 // K

_mesh = plsc.VectorSubcoreMesh(core_axis_name="c", subcore_axis_name="s")
_sc_params = pltpu.CompilerParams()
if "needs_layout_passes" in pltpu.CompilerParams.__dataclass_fields__:
    _sc_params = dataclasses.replace(_sc_params, needs_layout_passes=False)
_f32 = jnp.float32
_i32 = jnp.int32


def _zeros16():
    return jnp.zeros((16,), _f32)


@functools.partial(
    pl.kernel,
    mesh=_mesh,
    compiler_params=_sc_params,
    out_type=[jax.ShapeDtypeStruct((NC, NPAD), _f32),
              jax.ShapeDtypeStruct((NC, NPAD), _f32)],
    scratch_types=[pltpu.VMEM((K,), _i32),
                   pltpu.VMEM((K,), _f32),
                   pltpu.VMEM((K,), _f32),
                   pltpu.VMEM((640,), _f32),
                   pltpu.VMEM_SHARED((NPAD,), _f32),
                   pltpu.VMEM_SHARED((NPAD,), _f32)],
)
def _sc_deg_wsum(src_hbm, w_hbm, deg_out, wsum_out,
                 sidx, wv, ones, zb, deg_sh, wsum_sh):
    c = lax.axis_index("c")
    s = lax.axis_index("s")
    for i in range(40):
        zb[pl.ds(i * 16, 16)] = _zeros16()
    for i in range(K // 16):
        ones[pl.ds(i * 16, 16)] = jnp.ones((16,), _f32)
    pltpu.sync_copy(zb, deg_sh.at[pl.ds(s * 640, 640)])
    pltpu.sync_copy(zb, wsum_sh.at[pl.ds(s * 640, 640)])
    plsc.subcore_barrier()

    tile_base = (c * NS + s) * EPT

    @pl.loop(0, CH)
    def _(j):
        base = tile_base + j * K
        pltpu.sync_copy(src_hbm.at[pl.ds(base, K)], sidx)
        pltpu.sync_copy(w_hbm.at[pl.ds(base, K)], wv)
        pltpu.sync_copy(ones, deg_sh.at[sidx], add=True)
        pltpu.sync_copy(wv, wsum_sh.at[sidx], add=True)

    plsc.subcore_barrier()
    pltpu.sync_copy(deg_sh.at[pl.ds(s * 640, 640)], zb)
    pltpu.sync_copy(zb, deg_out.at[c, pl.ds(s * 640, 640)])
    pltpu.sync_copy(wsum_sh.at[pl.ds(s * 640, 640)], zb)
    pltpu.sync_copy(zb, wsum_out.at[c, pl.ds(s * 640, 640)])


def _tcb_body(degp, wsump, x, xs, norm_o, svn_o):
    deg = degp[0] + degp[1]
    wsum = wsump[0] + wsump[1]
    normf = 1.0 / jnp.maximum(deg, 1.0)
    norm = normf[:N, :]
    norm_o[...] = norm
    svn_o[...] = norm * wsum[:N, :]
    xs[...] = x[...] * norm


def _tc_norm_scale(degp, wsump, x):
    return pl.pallas_call(
        _tcb_body,
        out_shape=[jax.ShapeDtypeStruct((N, D), _f32),
                   jax.ShapeDtypeStruct((N, 1), _f32),
                   jax.ShapeDtypeStruct((N, 1), _f32)],
    )(degp, wsump, x)


@functools.partial(
    pl.kernel,
    mesh=_mesh,
    compiler_params=_sc_params,
    out_type=jax.ShapeDtypeStruct((NC, NPAD, D), _f32),
    scratch_types=[pltpu.VMEM((K,), _i32),
                   pltpu.VMEM((K,), _i32),
                   pltpu.VMEM((K,), _f32),
                   pltpu.VMEM((K, D), _f32),
                   pltpu.VMEM((128, D), _f32),
                   pltpu.VMEM_SHARED((NPAD, D), _f32)],
)
def _sc_agg(src_hbm, dst_hbm, w_hbm, xs_hbm, agg_out,
            sidx, didx, wv, rows, zrows, agg_sh):
    c = lax.axis_index("c")
    s = lax.axis_index("s")
    for r in range(128):
        for cc in range(D // 16):
            zrows[r, pl.ds(cc * 16, 16)] = _zeros16()
    row0 = s * (NPAD // NS)
    for b in range(5):
        pltpu.sync_copy(zrows, agg_sh.at[pl.ds(row0 + b * 128, 128)])
    plsc.subcore_barrier()

    tile_base = (c * NS + s) * EPT

    @pl.loop(0, CH)
    def _(j):
        base = tile_base + j * K
        pltpu.sync_copy(src_hbm.at[pl.ds(base, K)], sidx)
        pltpu.sync_copy(dst_hbm.at[pl.ds(base, K)], didx)
        pltpu.sync_copy(w_hbm.at[pl.ds(base, K)], wv)
        pltpu.sync_copy(xs_hbm.at[sidx], rows)
        for g in range(K // 16):
            wreg = wv[pl.ds(g * 16, 16)]
            for kk in range(16):
                k = g * 16 + kk
                ws = wreg.at[jnp.full((16,), kk, _i32)].get(
                    mode="promise_in_bounds")
                for cc in range(D // 16):
                    sl = pl.ds(cc * 16, 16)
                    rows[k, sl] = rows[k, sl] * ws
        pltpu.sync_copy(rows, agg_sh.at[didx], add=True)

    plsc.subcore_barrier()
    for b in range(5):
        pltpu.sync_copy(agg_sh.at[pl.ds(row0 + b * 128, 128)], zrows)
        pltpu.sync_copy(zrows, agg_out.at[c, pl.ds(row0 + b * 128, 128)])


def _tcd_body(aggp, norm, svn, W0, b0, W1, b1, Wp, bp, Wv, bv,
              pn_o, V_o, c_o):
    agg = (aggp[0] + aggp[1])[:N, :]
    rst0 = jnp.maximum(
        jnp.dot(agg, W0[...], preferred_element_type=_f32) + b0[...], 0.0)
    t = jnp.dot(rst0, W1[...], preferred_element_type=_f32)
    p = jnp.dot(t, Wp[...], preferred_element_type=_f32)
    pv = jnp.dot(t, Wv[...], preferred_element_type=_f32)
    pn_o[...] = p * norm[...]
    bWv = jnp.dot(b1[...], Wv[...], preferred_element_type=_f32)
    bWp = jnp.dot(b1[...], Wp[...], preferred_element_type=_f32)
    V_o[...] = jnp.sum(svn[...] * pv, axis=0, keepdims=True) / N + bWv + bv[...]
    c_o[...] = jnp.broadcast_to(bWp + bp[...], (1, 16))


def _tc_dense(aggp, norm, svn, W0, b0, W1, b1, Wp, bp, Wv, bv):
    return pl.pallas_call(
        _tcd_body,
        out_shape=[jax.ShapeDtypeStruct((N, 1), _f32),
                   jax.ShapeDtypeStruct((1, 1), _f32),
                   jax.ShapeDtypeStruct((1, 16), _f32)],
    )(aggp, norm, svn, W0, b0, W1, b1, Wp, bp, Wv, bv)


@functools.partial(
    pl.kernel,
    mesh=_mesh,
    compiler_params=_sc_params,
    out_type=jax.ShapeDtypeStruct((NPAD,), _f32),
    scratch_types=[pltpu.VMEM((K,), _i32),
                   pltpu.VMEM((K,), _i32),
                   pltpu.VMEM((K,), _f32),
                   pltpu.VMEM((K,), _f32),
                   pltpu.VMEM((640,), _f32),
                   pltpu.VMEM((16,), _f32),
                   pltpu.VMEM_SHARED((NPAD,), _f32),
                   pltpu.VMEM_SHARED((NPAD,), _f32)],
)
def _sc_pi(src_hbm, dst_hbm, w_hbm, pn_hbm, c_hbm, pi_out,
           sidx, didx, wv, pb, zb, cb, pn_sh, pi_sh):
    c = lax.axis_index("c")
    s = lax.axis_index("s")

    @pl.when(c == 0)
    def _():
        for i in range(40):
            zb[pl.ds(i * 16, 16)] = _zeros16()
        pltpu.sync_copy(zb, pi_sh.at[pl.ds(s * 640, 640)])
        pltpu.sync_copy(pn_hbm.at[pl.ds(s * 640, 640)], zb)
        pltpu.sync_copy(zb, pn_sh.at[pl.ds(s * 640, 640)])
        pltpu.sync_copy(c_hbm, cb)
        plsc.subcore_barrier()

        tile_base = s * EPT1

        @pl.loop(0, CH1)
        def _(j):
            base = tile_base + j * K
            pltpu.sync_copy(src_hbm.at[pl.ds(base, K)], sidx)
            pltpu.sync_copy(dst_hbm.at[pl.ds(base, K)], didx)
            pltpu.sync_copy(w_hbm.at[pl.ds(base, K)], wv)
            pltpu.sync_copy(pn_sh.at[sidx], pb)
            for i in range(K // 16):
                sl = pl.ds(i * 16, 16)
                pb[sl] = pb[sl] * wv[sl]
            pltpu.sync_copy(pb, pi_sh.at[didx], add=True)

        plsc.subcore_barrier()
        cs = cb[pl.ds(0, 16)]
        csplat = cs.at[jnp.zeros((16,), _i32)].get(mode="promise_in_bounds")
        pltpu.sync_copy(pi_sh.at[pl.ds(s * 640, 640)], zb)
        for i in range(40):
            sl = pl.ds(i * 16, 16)
            zb[sl] = zb[sl] + csplat
        pltpu.sync_copy(zb, pi_out.at[pl.ds(s * 640, 640)])


def kernel(x, edge_index, w, W0, b0, W1, b1, Wp, bp, Wv, bv):
    src = edge_index[0]
    dst = edge_index[1]

    degp, wsump = _sc_deg_wsum(src, w)
    degp = degp.reshape(NC, NPAD, 1)
    wsump = wsump.reshape(NC, NPAD, 1)
    xs, norm, svn = _tc_norm_scale(degp, wsump, x)
    aggp = _sc_agg(src, dst, w, xs)
    pn, V, c16 = _tc_dense(aggp, norm, svn,
                           W0, b0.reshape(1, D), W1, b1.reshape(1, D),
                           Wp, bp.reshape(1, 1), Wv, bv.reshape(1, 1))
    pn_pad = jnp.pad(pn[:, 0], (0, NPAD - N))
    pi_pad = _sc_pi(src, dst, w, pn_pad, c16[0])
    PI = pi_pad[:N].reshape(N, 1)
    return (PI, V)


# speedup vs baseline: 4.5362x; 1.2336x over previous
import dataclasses
import functools

import jax
import jax.numpy as jnp
from jax import lax
from jax.experimental import pallas as pl
from jax.experimental.pallas import tpu as pltpu
from jax.experimental.pallas import tpu_sc as plsc

N = 10000
E = 320000
D = 128
NPAD = 10240
NC = 2
NS = 16
K = 80
EPT = E // (NC * NS)
CH = EPT // K
EPT1 = E // NS
CH1 = EPT1 // K

_mesh = plsc.VectorSubcoreMesh(core_axis_name="c", subcore_axis_name="s")
_sc_params = pltpu.CompilerParams()
if "needs_layout_passes" in pltpu.CompilerParams.__dataclass_fields__:
    _sc_params = dataclasses.replace(_sc_params, needs_layout_passes=False)
_f32 = jnp.float32
_i32 = jnp.int32


def _zeros16():
    return jnp.zeros((16,), _f32)


@functools.partial(
    pl.kernel,
    mesh=_mesh,
    compiler_params=_sc_params,
    out_type=[jax.ShapeDtypeStruct((NC, NPAD), _f32),
              jax.ShapeDtypeStruct((NC, NPAD), _f32)],
    scratch_types=[pltpu.VMEM((K,), _i32),
                   pltpu.VMEM((K,), _f32),
                   pltpu.VMEM((K,), _f32),
                   pltpu.VMEM((640,), _f32),
                   pltpu.VMEM_SHARED((NPAD,), _f32),
                   pltpu.VMEM_SHARED((NPAD,), _f32)],
)
def _sc_deg_wsum(src_hbm, w_hbm, deg_out, wsum_out,
                 sidx, wv, ones, zb, deg_sh, wsum_sh):
    c = lax.axis_index("c")
    s = lax.axis_index("s")
    for i in range(40):
        zb[pl.ds(i * 16, 16)] = _zeros16()
    for i in range(K // 16):
        ones[pl.ds(i * 16, 16)] = jnp.ones((16,), _f32)
    pltpu.sync_copy(zb, deg_sh.at[pl.ds(s * 640, 640)])
    pltpu.sync_copy(zb, wsum_sh.at[pl.ds(s * 640, 640)])
    plsc.subcore_barrier()

    tile_base = (c * NS + s) * EPT

    @pl.loop(0, CH)
    def _(j):
        base = tile_base + j * K
        pltpu.sync_copy(src_hbm.at[pl.ds(base, K)], sidx)
        pltpu.sync_copy(w_hbm.at[pl.ds(base, K)], wv)
        pltpu.sync_copy(ones, deg_sh.at[sidx], add=True)
        pltpu.sync_copy(wv, wsum_sh.at[sidx], add=True)

    plsc.subcore_barrier()
    pltpu.sync_copy(deg_sh.at[pl.ds(s * 640, 640)], zb)
    pltpu.sync_copy(zb, deg_out.at[c, pl.ds(s * 640, 640)])
    pltpu.sync_copy(wsum_sh.at[pl.ds(s * 640, 640)], zb)
    pltpu.sync_copy(zb, wsum_out.at[c, pl.ds(s * 640, 640)])


def _tcb_body(degp, wsump, x, xs, norm_o, svn_o):
    deg = degp[0] + degp[1]
    wsum = wsump[0] + wsump[1]
    normf = 1.0 / jnp.maximum(deg, 1.0)
    norm = normf[:N, :]
    norm_o[...] = norm
    svn_o[...] = norm * wsum[:N, :]
    xs[...] = x[...] * norm


def _tc_norm_scale(degp, wsump, x):
    return pl.pallas_call(
        _tcb_body,
        out_shape=[jax.ShapeDtypeStruct((N, D), _f32),
                   jax.ShapeDtypeStruct((N, 1), _f32),
                   jax.ShapeDtypeStruct((N, 1), _f32)],
    )(degp, wsump, x)


@functools.partial(
    pl.kernel,
    mesh=_mesh,
    compiler_params=_sc_params,
    out_type=jax.ShapeDtypeStruct((NC, NPAD, D), _f32),
    scratch_types=[pltpu.VMEM((K,), _i32),
                   pltpu.VMEM((K,), _i32),
                   pltpu.VMEM((K,), _f32),
                   pltpu.VMEM((K, D), _f32),
                   pltpu.VMEM((128, D), _f32),
                   pltpu.VMEM_SHARED((NPAD, D), _f32)],
)
def _sc_agg(src_hbm, dst_hbm, w_hbm, xs_hbm, agg_out,
            sidx, didx, wv, rows, zrows, agg_sh):
    c = lax.axis_index("c")
    s = lax.axis_index("s")
    for r in range(128):
        for cc in range(D // 16):
            zrows[r, pl.ds(cc * 16, 16)] = _zeros16()
    row0 = s * (NPAD // NS)
    for b in range(5):
        pltpu.sync_copy(zrows, agg_sh.at[pl.ds(row0 + b * 128, 128)])
    plsc.subcore_barrier()

    tile_base = (c * NS + s) * EPT

    @pl.loop(0, CH)
    def _(j):
        base = tile_base + j * K
        pltpu.sync_copy(src_hbm.at[pl.ds(base, K)], sidx)
        pltpu.sync_copy(dst_hbm.at[pl.ds(base, K)], didx)
        pltpu.sync_copy(w_hbm.at[pl.ds(base, K)], wv)
        pltpu.sync_copy(xs_hbm.at[sidx], rows)
        for g in range(K // 16):
            wreg = wv[pl.ds(g * 16, 16)]
            for kk in range(16):
                k = g * 16 + kk
                ws = wreg.at[jnp.full((16,), kk, _i32)].get(
                    mode="promise_in_bounds")
                for cc in range(D // 16):
                    sl = pl.ds(cc * 16, 16)
                    rows[k, sl] = rows[k, sl] * ws
        pltpu.sync_copy(rows, agg_sh.at[didx], add=True)

    plsc.subcore_barrier()
    for b in range(5):
        pltpu.sync_copy(agg_sh.at[pl.ds(row0 + b * 128, 128)], zrows)
        pltpu.sync_copy(zrows, agg_out.at[c, pl.ds(row0 + b * 128, 128)])


def _tcd_body(aggp, norm, svn, W0, b0, W1, b1, Wp, bp, Wv, bv,
              pn_o, V_o, c_o):
    agg = (aggp[0] + aggp[1])[:N, :]
    rst0 = jnp.maximum(
        jnp.dot(agg, W0[...], preferred_element_type=_f32) + b0[...], 0.0)
    t = jnp.dot(rst0, W1[...], preferred_element_type=_f32)
    p = jnp.dot(t, Wp[...], preferred_element_type=_f32)
    pv = jnp.dot(t, Wv[...], preferred_element_type=_f32)
    pn_o[...] = p * norm[...]
    bWv = jnp.dot(b1[...], Wv[...], preferred_element_type=_f32)
    bWp = jnp.dot(b1[...], Wp[...], preferred_element_type=_f32)
    V_o[...] = jnp.sum(svn[...] * pv, axis=0, keepdims=True) / N + bWv + bv[...]
    c_o[...] = jnp.broadcast_to(bWp + bp[...], (1, 16))


def _tc_dense(aggp, norm, svn, W0, b0, W1, b1, Wp, bp, Wv, bv):
    return pl.pallas_call(
        _tcd_body,
        out_shape=[jax.ShapeDtypeStruct((N, 1), _f32),
                   jax.ShapeDtypeStruct((1, 1), _f32),
                   jax.ShapeDtypeStruct((1, 16), _f32)],
    )(aggp, norm, svn, W0, b0, W1, b1, Wp, bp, Wv, bv)


@functools.partial(
    pl.kernel,
    mesh=_mesh,
    compiler_params=_sc_params,
    out_type=jax.ShapeDtypeStruct((NC, NPAD), _f32),
    scratch_types=[pltpu.VMEM((K,), _i32),
                   pltpu.VMEM((K,), _i32),
                   pltpu.VMEM((K,), _f32),
                   pltpu.VMEM((K,), _f32),
                   pltpu.VMEM((640,), _f32),
                   pltpu.VMEM_SHARED((NPAD,), _f32),
                   pltpu.VMEM_SHARED((NPAD,), _f32)],
)
def _sc_pi(src_hbm, dst_hbm, w_hbm, pn_hbm, pi_out,
           sidx, didx, wv, pb, zb, pn_sh, pi_sh):
    c = lax.axis_index("c")
    s = lax.axis_index("s")
    for i in range(40):
        zb[pl.ds(i * 16, 16)] = _zeros16()
    pltpu.sync_copy(zb, pi_sh.at[pl.ds(s * 640, 640)])
    pltpu.sync_copy(pn_hbm.at[pl.ds(s * 640, 640)], zb)
    pltpu.sync_copy(zb, pn_sh.at[pl.ds(s * 640, 640)])
    plsc.subcore_barrier()

    tile_base = (c * NS + s) * EPT

    @pl.loop(0, CH)
    def _(j):
        base = tile_base + j * K
        pltpu.sync_copy(src_hbm.at[pl.ds(base, K)], sidx)
        pltpu.sync_copy(dst_hbm.at[pl.ds(base, K)], didx)
        pltpu.sync_copy(w_hbm.at[pl.ds(base, K)], wv)
        pltpu.sync_copy(pn_sh.at[sidx], pb)
        for i in range(K // 16):
            sl = pl.ds(i * 16, 16)
            pb[sl] = pb[sl] * wv[sl]
        pltpu.sync_copy(pb, pi_sh.at[didx], add=True)

    plsc.subcore_barrier()
    pltpu.sync_copy(pi_sh.at[pl.ds(s * 640, 640)], zb)
    pltpu.sync_copy(zb, pi_out.at[c, pl.ds(s * 640, 640)])


def kernel(x, edge_index, w, W0, b0, W1, b1, Wp, bp, Wv, bv):
    src = edge_index[0]
    dst = edge_index[1]

    degp, wsump = _sc_deg_wsum(src, w)
    degp = degp.reshape(NC, NPAD, 1)
    wsump = wsump.reshape(NC, NPAD, 1)
    xs, norm, svn = _tc_norm_scale(degp, wsump, x)
    aggp = _sc_agg(src, dst, w, xs)
    pn, V, c16 = _tc_dense(aggp, norm, svn,
                           W0, b0.reshape(1, D), W1, b1.reshape(1, D),
                           Wp, bp.reshape(1, 1), Wv, bv.reshape(1, 1))
    pn_pad = jnp.pad(pn[:, 0], (0, NPAD - N))
    pip = _sc_pi(src, dst, w, pn_pad)
    PI = (pip[0, :N] + pip[1, :N] + c16[0, 0]).reshape(N, 1)
    return (PI, V)


# speedup vs baseline: 8.3607x; 1.8431x over previous
import dataclasses
import functools

import jax
import jax.numpy as jnp
from jax import lax
from jax.experimental import pallas as pl
from jax.experimental.pallas import tpu as pltpu
from jax.experimental.pallas import tpu_sc as plsc

N = 10000
E = 320000
D = 128
NPAD = 10240
NC = 2
NS = 16
K = 80
EPT = E // (NC * NS)
CH = EPT // K
EPT1 = E // NS
CH1 = EPT1 // K

_mesh = plsc.VectorSubcoreMesh(core_axis_name="c", subcore_axis_name="s")
_sc_params = pltpu.CompilerParams()
if "needs_layout_passes" in pltpu.CompilerParams.__dataclass_fields__:
    _sc_params = dataclasses.replace(_sc_params, needs_layout_passes=False)
_f32 = jnp.float32
_i32 = jnp.int32


def _zeros16():
    return jnp.zeros((16,), _f32)


@functools.partial(
    pl.kernel,
    mesh=_mesh,
    compiler_params=_sc_params,
    out_type=[jax.ShapeDtypeStruct((NC, NPAD), _f32),
              jax.ShapeDtypeStruct((NC, NPAD), _f32)],
    scratch_types=[pltpu.VMEM((K,), _i32),
                   pltpu.VMEM((K,), _f32),
                   pltpu.VMEM((K,), _f32),
                   pltpu.VMEM((640,), _f32),
                   pltpu.VMEM_SHARED((NPAD,), _f32),
                   pltpu.VMEM_SHARED((NPAD,), _f32),
                   pltpu.SemaphoreType.DMA],
)
def _sc_deg_wsum(src_hbm, w_hbm, deg_out, wsum_out,
                 sidx, wv, ones, zb, deg_sh, wsum_sh, lsem):
    c = lax.axis_index("c")
    s = lax.axis_index("s")
    for i in range(40):
        zb[pl.ds(i * 16, 16)] = _zeros16()
    for i in range(K // 16):
        ones[pl.ds(i * 16, 16)] = jnp.ones((16,), _f32)
    pltpu.sync_copy(zb, deg_sh.at[pl.ds(s * 640, 640)])
    pltpu.sync_copy(zb, wsum_sh.at[pl.ds(s * 640, 640)])
    plsc.subcore_barrier()

    tile_base = (c * NS + s) * EPT

    @pl.loop(0, CH)
    def _(j):
        base = tile_base + j * K
        c1 = pltpu.async_copy(src_hbm.at[pl.ds(base, K)], sidx, lsem)
        c2 = pltpu.async_copy(w_hbm.at[pl.ds(base, K)], wv, lsem)
        c1.wait()
        c2.wait()
        pltpu.sync_copy(ones, deg_sh.at[sidx], add=True)
        pltpu.sync_copy(wv, wsum_sh.at[sidx], add=True)

    plsc.subcore_barrier()
    pltpu.sync_copy(deg_sh.at[pl.ds(s * 640, 640)], zb)
    pltpu.sync_copy(zb, deg_out.at[c, pl.ds(s * 640, 640)])
    pltpu.sync_copy(wsum_sh.at[pl.ds(s * 640, 640)], zb)
    pltpu.sync_copy(zb, wsum_out.at[c, pl.ds(s * 640, 640)])


def _tcb_body(degp, wsump, x, xs, norm_o, svn_o):
    deg = degp[0] + degp[1]
    wsum = wsump[0] + wsump[1]
    normf = 1.0 / jnp.maximum(deg, 1.0)
    norm = normf[:N, :]
    norm_o[...] = norm
    svn_o[...] = norm * wsum[:N, :]
    xs[...] = x[...] * norm


def _tc_norm_scale(degp, wsump, x):
    return pl.pallas_call(
        _tcb_body,
        out_shape=[jax.ShapeDtypeStruct((N, D), _f32),
                   jax.ShapeDtypeStruct((N, 1), _f32),
                   jax.ShapeDtypeStruct((N, 1), _f32)],
    )(degp, wsump, x)


@functools.partial(
    pl.kernel,
    mesh=_mesh,
    compiler_params=_sc_params,
    out_type=jax.ShapeDtypeStruct((NC, NPAD, D), _f32),
    scratch_types=[pltpu.VMEM((K,), _i32),
                   pltpu.VMEM((K,), _i32),
                   pltpu.VMEM((K,), _f32),
                   pltpu.VMEM((K,), _i32),
                   pltpu.VMEM((K,), _i32),
                   pltpu.VMEM((K,), _f32),
                   pltpu.VMEM((K, D), _f32),
                   pltpu.VMEM((K, D), _f32),
                   pltpu.VMEM((128, D), _f32),
                   pltpu.VMEM_SHARED((NPAD, D), _f32),
                   pltpu.SemaphoreType.DMA,
                   pltpu.SemaphoreType.DMA,
                   pltpu.SemaphoreType.DMA],
)
def _sc_agg(src_hbm, dst_hbm, w_hbm, xs_hbm, agg_out,
            sidx0, didx0, wv0, sidx1, didx1, wv1, rows0, rows1,
            zrows, agg_sh, lsem, gsem0, gsem1):
    c = lax.axis_index("c")
    s = lax.axis_index("s")
    for r in range(128):
        for cc in range(D // 16):
            zrows[r, pl.ds(cc * 16, 16)] = _zeros16()
    row0 = s * (NPAD // NS)
    for b in range(5):
        pltpu.sync_copy(zrows, agg_sh.at[pl.ds(row0 + b * 128, 128)])
    plsc.subcore_barrier()

    tile_base = (c * NS + s) * EPT

    def load_idx(j, si, di, wv_):
        base = tile_base + j * K
        c1 = pltpu.async_copy(src_hbm.at[pl.ds(base, K)], si, lsem)
        c2 = pltpu.async_copy(dst_hbm.at[pl.ds(base, K)], di, lsem)
        c3 = pltpu.async_copy(w_hbm.at[pl.ds(base, K)], wv_, lsem)
        c1.wait()
        c2.wait()
        c3.wait()

    def scale(rows, wv_):
        for g in range(K // 16):
            wreg = wv_[pl.ds(g * 16, 16)]
            for kk in range(16):
                k = g * 16 + kk
                ws = wreg.at[jnp.full((16,), kk, _i32)].get(
                    mode="promise_in_bounds")
                for cc in range(D // 16):
                    sl = pl.ds(cc * 16, 16)
                    rows[k, sl] = rows[k, sl] * ws

    load_idx(0, sidx0, didx0, wv0)
    pltpu.async_copy(xs_hbm.at[sidx0], rows0, gsem0)

    @pl.loop(0, CH // 2)
    def _(i):
        j0 = 2 * i
        load_idx(j0 + 1, sidx1, didx1, wv1)
        pltpu.make_async_copy(xs_hbm.at[sidx0], rows0, gsem0).wait()
        pltpu.async_copy(xs_hbm.at[sidx1], rows1, gsem1)
        scale(rows0, wv0)
        pltpu.sync_copy(rows0, agg_sh.at[didx0], add=True)

        load_idx(j0 + 2, sidx0, didx0, wv0)
        pltpu.make_async_copy(xs_hbm.at[sidx1], rows1, gsem1).wait()
        pltpu.async_copy(xs_hbm.at[sidx0], rows0, gsem0)
        scale(rows1, wv1)
        pltpu.sync_copy(rows1, agg_sh.at[didx1], add=True)

    pltpu.make_async_copy(xs_hbm.at[sidx0], rows0, gsem0).wait()
    scale(rows0, wv0)
    pltpu.sync_copy(rows0, agg_sh.at[didx0], add=True)

    plsc.subcore_barrier()
    for b in range(5):
        pltpu.sync_copy(agg_sh.at[pl.ds(row0 + b * 128, 128)], zrows)
        pltpu.sync_copy(zrows, agg_out.at[c, pl.ds(row0 + b * 128, 128)])


def _tcd_body(aggp, norm, svn, W0, b0, W1, b1, Wp, bp, Wv, bv,
              pn_o, V_o, c_o):
    agg = (aggp[0] + aggp[1])[:N, :]
    rst0 = jnp.maximum(
        jnp.dot(agg, W0[...], preferred_element_type=_f32) + b0[...], 0.0)
    t = jnp.dot(rst0, W1[...], preferred_element_type=_f32)
    p = jnp.dot(t, Wp[...], preferred_element_type=_f32)
    pv = jnp.dot(t, Wv[...], preferred_element_type=_f32)
    pn_o[...] = p * norm[...]
    bWv = jnp.dot(b1[...], Wv[...], preferred_element_type=_f32)
    bWp = jnp.dot(b1[...], Wp[...], preferred_element_type=_f32)
    V_o[...] = jnp.sum(svn[...] * pv, axis=0, keepdims=True) / N + bWv + bv[...]
    c_o[...] = jnp.broadcast_to(bWp + bp[...], (1, 16))


def _tc_dense(aggp, norm, svn, W0, b0, W1, b1, Wp, bp, Wv, bv):
    return pl.pallas_call(
        _tcd_body,
        out_shape=[jax.ShapeDtypeStruct((N, 1), _f32),
                   jax.ShapeDtypeStruct((1, 1), _f32),
                   jax.ShapeDtypeStruct((1, 16), _f32)],
    )(aggp, norm, svn, W0, b0, W1, b1, Wp, bp, Wv, bv)


@functools.partial(
    pl.kernel,
    mesh=_mesh,
    compiler_params=_sc_params,
    out_type=jax.ShapeDtypeStruct((NC, NPAD), _f32),
    scratch_types=[pltpu.VMEM((K,), _i32),
                   pltpu.VMEM((K,), _i32),
                   pltpu.VMEM((K,), _f32),
                   pltpu.VMEM((K,), _f32),
                   pltpu.VMEM((640,), _f32),
                   pltpu.VMEM_SHARED((NPAD,), _f32),
                   pltpu.VMEM_SHARED((NPAD,), _f32),
                   pltpu.SemaphoreType.DMA],
)
def _sc_pi(src_hbm, dst_hbm, w_hbm, pn_hbm, pi_out,
           sidx, didx, wv, pb, zb, pn_sh, pi_sh, lsem):
    c = lax.axis_index("c")
    s = lax.axis_index("s")
    for i in range(40):
        zb[pl.ds(i * 16, 16)] = _zeros16()
    pltpu.sync_copy(zb, pi_sh.at[pl.ds(s * 640, 640)])
    pltpu.sync_copy(pn_hbm.at[pl.ds(s * 640, 640)], zb)
    pltpu.sync_copy(zb, pn_sh.at[pl.ds(s * 640, 640)])
    plsc.subcore_barrier()

    tile_base = (c * NS + s) * EPT

    @pl.loop(0, CH)
    def _(j):
        base = tile_base + j * K
        c1 = pltpu.async_copy(src_hbm.at[pl.ds(base, K)], sidx, lsem)
        c2 = pltpu.async_copy(dst_hbm.at[pl.ds(base, K)], didx, lsem)
        c3 = pltpu.async_copy(w_hbm.at[pl.ds(base, K)], wv, lsem)
        c1.wait()
        c2.wait()
        c3.wait()
        pltpu.sync_copy(pn_sh.at[sidx], pb)
        for i in range(K // 16):
            sl = pl.ds(i * 16, 16)
            pb[sl] = pb[sl] * wv[sl]
        pltpu.sync_copy(pb, pi_sh.at[didx], add=True)

    plsc.subcore_barrier()
    pltpu.sync_copy(pi_sh.at[pl.ds(s * 640, 640)], zb)
    pltpu.sync_copy(zb, pi_out.at[c, pl.ds(s * 640, 640)])


def kernel(x, edge_index, w, W0, b0, W1, b1, Wp, bp, Wv, bv):
    src = edge_index[0]
    dst = edge_index[1]

    degp, wsump = _sc_deg_wsum(src, w)
    degp = degp.reshape(NC, NPAD, 1)
    wsump = wsump.reshape(NC, NPAD, 1)
    xs, norm, svn = _tc_norm_scale(degp, wsump, x)
    aggp = _sc_agg(src, dst, w, xs)
    pn, V, c16 = _tc_dense(aggp, norm, svn,
                           W0, b0.reshape(1, D), W1, b1.reshape(1, D),
                           Wp, bp.reshape(1, 1), Wv, bv.reshape(1, 1))
    pn_pad = jnp.pad(pn[:, 0], (0, NPAD - N))
    pip = _sc_pi(src, dst, w, pn_pad)
    PI = (pip[0, :N] + pip[1, :N] + c16[0, 0]).reshape(N, 1)
    return (PI, V)
